# Initial kernel scaffold; baseline (speedup 1.0000x reference)
import jax
import jax.numpy as jnp
from jax.experimental import pallas as pl


def kernel(x, in_degree, out_degree, link, length, entity_table, in_deg_table, out_deg_table, rel_table, ffn_W, ln_gamma, ln_beta, graph_token):
    raise NotImplementedError("write your pallas kernel here")


# speedup vs baseline: 1.6292x; 1.6292x over previous
import functools

import jax
import jax.numpy as jnp
from jax import lax
from jax.experimental import pallas as pl
from jax.experimental.pallas import tpu as pltpu
from jax.experimental.pallas import tpu_sc as plsc

B, N, K, H = 16, 1024, 4, 128
R = 512
T = B * N
NC, NS = 2, 16
NW = NC * NS
TPW = T // NW
TOK_CH = 128
IDX_CH = TOK_CH * K
N_CH = TPW // TOK_CH
GPC = IDX_CH // 128


def _ent_gather_body(idx_hbm, table_hbm, out_hbm, idx_v, rows_v, out_v, sem):
    wid = lax.axis_index("s") * NC + lax.axis_index("c")
    pltpu.sync_copy(idx_hbm.at[pl.ds(wid * (TPW * K // 128), TPW * K // 128)],
                    idx_v)
    for c in range(N_CH):
        tok_base = wid * TPW + c * TOK_CH
        cps = []
        for g in range(GPC):
            cps.append(pltpu.async_copy(
                table_hbm.at[idx_v.at[c * GPC + g]],
                rows_v.at[pl.ds(g * 128, 128)], sem))
        for cp in cps:
            cp.wait()

        def red(t, _):
            for h in range(H // 16):
                s = pl.ds(h * 16, 16)
                out_v[t, s] = (rows_v[4 * t, s] + rows_v[4 * t + 1, s]
                               + rows_v[4 * t + 2, s] + rows_v[4 * t + 3, s])
            return 0
        lax.fori_loop(0, TOK_CH, red, 0)
        pltpu.sync_copy(out_v, out_hbm.at[pl.ds(tok_base, TOK_CH)])


_ent_gather = functools.partial(
    pl.kernel,
    mesh=plsc.VectorSubcoreMesh(core_axis_name="c", subcore_axis_name="s"),
    out_type=jax.ShapeDtypeStruct((T, H), jnp.float32),
    scratch_types=[
        pltpu.VMEM((TPW * K // 128, 128), jnp.int32),
        pltpu.VMEM((IDX_CH, H), jnp.float32),
        pltpu.VMEM((TOK_CH, H), jnp.float32),
        pltpu.SemaphoreType.DMA,
    ],
)(_ent_gather_body)


def _w2_body(a_ref, b_ref, o_ref):
    o_ref[...] = jnp.dot(a_ref[...], b_ref[...],
                         preferred_element_type=jnp.float32)


def _tc_body(link_ref, ent_ref, w2_ref, g_ref, b_ref, out_ref):
    ls = (link_ref[:, 0, :] + link_ref[:, 1, :]
          + link_ref[:, 2, :] + link_ref[:, 3, :])
    acc = jnp.dot(ls, w2_ref[...], preferred_element_type=jnp.float32)
    acc = acc + ent_ref[...]
    mu = jnp.mean(acc, axis=-1, keepdims=True)
    d = acc - mu
    var = jnp.mean(d * d, axis=-1, keepdims=True)
    out_ref[...] = d * lax.rsqrt(var + 1e-6) * g_ref[...] + b_ref[...]


TBLK = 512


def kernel(x, in_degree, out_degree, link, length, entity_table,
           in_deg_table, out_deg_table, rel_table, ffn_W,
           ln_gamma, ln_beta, graph_token):
    idx = x.astype(jnp.int32).reshape(T * K // 128, 128)
    ent = _ent_gather(idx, entity_table)

    w2 = pl.pallas_call(
        _w2_body,
        out_shape=jax.ShapeDtypeStruct((R, H), jnp.float32),
    )(rel_table, ffn_W)

    link_flat = link.reshape(T, K, R)
    g2 = ln_gamma.reshape(1, H)
    b2 = ln_beta.reshape(1, H)
    feat = pl.pallas_call(
        _tc_body,
        grid=(T // TBLK,),
        in_specs=[
            pl.BlockSpec((TBLK, K, R), lambda i: (i, 0, 0)),
            pl.BlockSpec((TBLK, H), lambda i: (i, 0)),
            pl.BlockSpec((R, H), lambda i: (0, 0)),
            pl.BlockSpec((1, H), lambda i: (0, 0)),
            pl.BlockSpec((1, H), lambda i: (0, 0)),
        ],
        out_specs=pl.BlockSpec((TBLK, H), lambda i: (i, 0)),
        out_shape=jax.ShapeDtypeStruct((T, H), jnp.float32),
    )(link_flat, ent, w2, g2, b2)

    gt = jnp.broadcast_to(graph_token[None, :, :], (B, 1, H))
    return jnp.concatenate([gt, feat.reshape(B, N, H)], axis=1)


# speedup vs baseline: 2.0232x; 1.2418x over previous
import functools

import jax
import jax.numpy as jnp
from jax import lax
from jax.experimental import pallas as pl
from jax.experimental.pallas import tpu as pltpu
from jax.experimental.pallas import tpu_sc as plsc

B, N, K, H = 16, 1024, 4, 128
R = 512
T = B * N
NC, NS = 2, 16
NW = NC * NS
TPW = T // NW
TOK_CH = 128
IDX_CH = TOK_CH * K
N_CH = TPW // TOK_CH
GPC = IDX_CH // 128


def _ent_gather_body(idx_hbm, table_hbm, out_hbm, idx_v, rows_v, out_v, sem):
    wid = lax.axis_index("s") * NC + lax.axis_index("c")
    pltpu.sync_copy(idx_hbm.at[pl.ds(wid * (TPW * K // 128), TPW * K // 128)],
                    idx_v)
    for c in range(N_CH):
        tok_base = wid * TPW + c * TOK_CH
        cps = []
        for g in range(GPC):
            cps.append(pltpu.async_copy(
                table_hbm.at[idx_v.at[c * GPC + g]],
                rows_v.at[pl.ds(g * 128, 128)], sem))
        for cp in cps:
            cp.wait()

        def red(t, _):
            for h in range(H // 16):
                s = pl.ds(h * 16, 16)
                out_v[t, s] = (rows_v[4 * t, s] + rows_v[4 * t + 1, s]
                               + rows_v[4 * t + 2, s] + rows_v[4 * t + 3, s])
            return 0
        lax.fori_loop(0, TOK_CH, red, 0)
        pltpu.sync_copy(out_v, out_hbm.at[pl.ds(tok_base, TOK_CH)])


_ent_gather = functools.partial(
    pl.kernel,
    mesh=plsc.VectorSubcoreMesh(core_axis_name="c", subcore_axis_name="s"),
    out_type=jax.ShapeDtypeStruct((T, H), jnp.float32),
    scratch_types=[
        pltpu.VMEM((TPW * K // 128, 128), jnp.int32),
        pltpu.VMEM((IDX_CH, H), jnp.float32),
        pltpu.VMEM((TOK_CH, H), jnp.float32),
        pltpu.SemaphoreType.DMA,
    ],
)(_ent_gather_body)


def _w2_body(a_ref, b_ref, o_ref):
    o_ref[...] = jnp.dot(a_ref[...], b_ref[...],
                         preferred_element_type=jnp.float32)


def _rel_body(link_ref, w2_ref, out_ref):
    ls = (link_ref[:, 0, :] + link_ref[:, 1, :]
          + link_ref[:, 2, :] + link_ref[:, 3, :])
    out_ref[...] = jnp.dot(ls, w2_ref[...],
                           preferred_element_type=jnp.float32)


def _fin_body(rel_ref, ent_ref, gt_ref, g_ref, b_ref, out_ref):
    acc = rel_ref[0] + ent_ref[0]
    mu = jnp.mean(acc, axis=-1, keepdims=True)
    d = acc - mu
    var = jnp.mean(d * d, axis=-1, keepdims=True)
    y = d * lax.rsqrt(var + 1e-6) * g_ref[...] + b_ref[...]
    out_ref[0, 0:1, :] = gt_ref[...]
    out_ref[0, 1:, :] = y


TBLK = 512


def kernel(x, in_degree, out_degree, link, length, entity_table,
           in_deg_table, out_deg_table, rel_table, ffn_W,
           ln_gamma, ln_beta, graph_token):
    idx = x.astype(jnp.int32).reshape(T * K // 128, 128)
    ent = _ent_gather(idx, entity_table)

    w2 = pl.pallas_call(
        _w2_body,
        out_shape=jax.ShapeDtypeStruct((R, H), jnp.float32),
    )(rel_table, ffn_W)

    link_flat = link.reshape(T, K, R)
    rel = pl.pallas_call(
        _rel_body,
        grid=(T // TBLK,),
        in_specs=[
            pl.BlockSpec((TBLK, K, R), lambda i: (i, 0, 0)),
            pl.BlockSpec((R, H), lambda i: (0, 0)),
        ],
        out_specs=pl.BlockSpec((TBLK, H), lambda i: (i, 0)),
        out_shape=jax.ShapeDtypeStruct((T, H), jnp.float32),
    )(link_flat, w2)

    g2 = ln_gamma.reshape(1, H)
    b2 = ln_beta.reshape(1, H)
    out = pl.pallas_call(
        _fin_body,
        grid=(B,),
        in_specs=[
            pl.BlockSpec((1, N, H), lambda i: (i, 0, 0)),
            pl.BlockSpec((1, N, H), lambda i: (i, 0, 0)),
            pl.BlockSpec((1, H), lambda i: (0, 0)),
            pl.BlockSpec((1, H), lambda i: (0, 0)),
            pl.BlockSpec((1, H), lambda i: (0, 0)),
        ],
        out_specs=pl.BlockSpec((1, N + 1, H), lambda i: (i, 0, 0)),
        out_shape=jax.ShapeDtypeStruct((B, N + 1, H), jnp.float32),
    )(rel.reshape(B, N, H), ent.reshape(B, N, H), graph_token, g2, b2)
    return out


# speedup vs baseline: 2.0628x; 1.0196x over previous
import functools

import jax
import jax.numpy as jnp
from jax import lax
from jax.experimental import pallas as pl
from jax.experimental.pallas import tpu as pltpu
from jax.experimental.pallas import tpu_sc as plsc

B, N, K, H = 16, 1024, 4, 128
R = 512
T = B * N
NC, NS = 2, 16
NW = NC * NS
TPW = T // NW
TOK_CH = 128
IDX_CH = TOK_CH * K
N_CH = TPW // TOK_CH
GPC = IDX_CH // 128


def _ent_gather_body(idx_hbm, table_hbm, out_hbm, idx_v, rows_v, out_v, sem):
    wid = lax.axis_index("s") * NC + lax.axis_index("c")
    pltpu.sync_copy(idx_hbm.at[pl.ds(wid * (TPW * K // 128), TPW * K // 128)],
                    idx_v)
    for c in range(N_CH):
        tok_base = wid * TPW + c * TOK_CH
        cps = []
        for g in range(GPC):
            cps.append(pltpu.async_copy(
                table_hbm.at[idx_v.at[c * GPC + g]],
                rows_v.at[pl.ds(g * 128, 128)], sem))
        for cp in cps:
            cp.wait()

        def red(t, _):
            for h in range(H // 16):
                s = pl.ds(h * 16, 16)
                out_v[t, s] = (rows_v[4 * t, s] + rows_v[4 * t + 1, s]
                               + rows_v[4 * t + 2, s] + rows_v[4 * t + 3, s])
            return 0
        lax.fori_loop(0, TOK_CH, red, 0)
        pltpu.sync_copy(out_v, out_hbm.at[pl.ds(tok_base, TOK_CH)])


_ent_gather = functools.partial(
    pl.kernel,
    mesh=plsc.VectorSubcoreMesh(core_axis_name="c", subcore_axis_name="s"),
    out_type=jax.ShapeDtypeStruct((T, H), jnp.float32),
    scratch_types=[
        pltpu.VMEM((TPW * K // 128, 128), jnp.int32),
        pltpu.VMEM((IDX_CH, H), jnp.float32),
        pltpu.VMEM((TOK_CH, H), jnp.float32),
        pltpu.SemaphoreType.DMA,
    ],
)(_ent_gather_body)


def _rel_body(rt_ref, fw_ref, link_ref, out_ref, w2_ref):
    @pl.when(pl.program_id(0) == 0)
    def _():
        w2_ref[...] = jnp.dot(rt_ref[...], fw_ref[...],
                              preferred_element_type=jnp.float32)

    ls = (link_ref[:, 0, :] + link_ref[:, 1, :]
          + link_ref[:, 2, :] + link_ref[:, 3, :])
    out_ref[...] = jnp.dot(ls, w2_ref[...],
                           preferred_element_type=jnp.float32
                           ).astype(jnp.bfloat16)


def _fin_body(rel_ref, ent_ref, gt_ref, g_ref, b_ref, out_ref):
    acc = rel_ref[0].astype(jnp.float32) + ent_ref[0]
    mu = jnp.mean(acc, axis=-1, keepdims=True)
    d = acc - mu
    var = jnp.mean(d * d, axis=-1, keepdims=True)
    y = d * lax.rsqrt(var + 1e-6) * g_ref[...] + b_ref[...]
    out_ref[0, 0:1, :] = gt_ref[...]
    out_ref[0, 1:, :] = y


TBLK = 512


def kernel(x, in_degree, out_degree, link, length, entity_table,
           in_deg_table, out_deg_table, rel_table, ffn_W,
           ln_gamma, ln_beta, graph_token):
    idx = x.astype(jnp.int32).reshape(T * K // 128, 128)
    ent = _ent_gather(idx, entity_table)

    link_flat = link.reshape(T, K, R)
    rel = pl.pallas_call(
        _rel_body,
        grid=(T // TBLK,),
        in_specs=[
            pl.BlockSpec((R, H), lambda i: (0, 0)),
            pl.BlockSpec((H, H), lambda i: (0, 0)),
            pl.BlockSpec((TBLK, K, R), lambda i: (i, 0, 0)),
        ],
        out_specs=pl.BlockSpec((TBLK, H), lambda i: (i, 0)),
        out_shape=jax.ShapeDtypeStruct((T, H), jnp.bfloat16),
        scratch_shapes=[pltpu.VMEM((R, H), jnp.float32)],
    )(rel_table, ffn_W, link_flat)

    g2 = ln_gamma.reshape(1, H)
    b2 = ln_beta.reshape(1, H)
    out = pl.pallas_call(
        _fin_body,
        grid=(B,),
        in_specs=[
            pl.BlockSpec((1, N, H), lambda i: (i, 0, 0)),
            pl.BlockSpec((1, N, H), lambda i: (i, 0, 0)),
            pl.BlockSpec((1, H), lambda i: (0, 0)),
            pl.BlockSpec((1, H), lambda i: (0, 0)),
            pl.BlockSpec((1, H), lambda i: (0, 0)),
        ],
        out_specs=pl.BlockSpec((1, N + 1, H), lambda i: (i, 0, 0)),
        out_shape=jax.ShapeDtypeStruct((B, N + 1, H), jnp.float32),
    )(rel.reshape(B, N, H), ent.reshape(B, N, H), graph_token, g2, b2)
    return out


# speedup vs baseline: 2.0709x; 1.0039x over previous
import functools

import jax
import jax.numpy as jnp
from jax import lax
from jax.experimental import pallas as pl
from jax.experimental.pallas import tpu as pltpu
from jax.experimental.pallas import tpu_sc as plsc

B, N, K, H = 16, 1024, 4, 128
R = 512
T = B * N
NC, NS = 2, 16
NW = NC * NS
TPW = T // NW
TOK_CH = 128
IDX_CH = TOK_CH * K
N_CH = TPW // TOK_CH
GPC = IDX_CH // 128


def _ent_gather_body(idx_hbm, table_hbm, out_hbm, idx_v, rows_v, out_v, sem):
    wid = lax.axis_index("s") * NC + lax.axis_index("c")
    pltpu.sync_copy(idx_hbm.at[pl.ds(wid * (TPW * K // 128), TPW * K // 128)],
                    idx_v)
    for c in range(N_CH):
        tok_base = wid * TPW + c * TOK_CH
        cps = []
        for g in range(GPC):
            cps.append(pltpu.async_copy(
                table_hbm.at[idx_v.at[c * GPC + g]],
                rows_v.at[pl.ds(g * 128, 128)], sem))
        for cp in cps:
            cp.wait()

        def red(t, _):
            for h in range(H // 16):
                s = pl.ds(h * 16, 16)
                out_v[t, s] = (rows_v[4 * t, s] + rows_v[4 * t + 1, s]
                               + rows_v[4 * t + 2, s] + rows_v[4 * t + 3, s])
            return 0
        lax.fori_loop(0, TOK_CH, red, 0)
        pltpu.sync_copy(out_v, out_hbm.at[pl.ds(tok_base, TOK_CH)])


_ent_gather = functools.partial(
    pl.kernel,
    mesh=plsc.VectorSubcoreMesh(core_axis_name="c", subcore_axis_name="s"),
    out_type=jax.ShapeDtypeStruct((T, H), jnp.float32),
    scratch_types=[
        pltpu.VMEM((TPW * K // 128, 128), jnp.int32),
        pltpu.VMEM((IDX_CH, H), jnp.float32),
        pltpu.VMEM((TOK_CH, H), jnp.float32),
        pltpu.SemaphoreType.DMA,
    ],
)(_ent_gather_body)


def _rel_body(rt_ref, fw_ref, link_ref, out_ref, w2_ref):
    @pl.when(pl.program_id(0) == 0)
    def _():
        w2_ref[...] = jnp.dot(rt_ref[...], fw_ref[...],
                              preferred_element_type=jnp.float32)

    ls = (link_ref[:, 0, :] + link_ref[:, 1, :]
          + link_ref[:, 2, :] + link_ref[:, 3, :])
    out_ref[...] = jnp.dot(ls, w2_ref[...],
                           preferred_element_type=jnp.float32
                           ).astype(jnp.bfloat16)


def _fin_body(rel_ref, ent_ref, gt_ref, g_ref, b_ref, out_ref):
    acc = rel_ref[0].astype(jnp.float32) + ent_ref[0]
    mu = jnp.mean(acc, axis=-1, keepdims=True)
    d = acc - mu
    var = jnp.mean(d * d, axis=-1, keepdims=True)
    y = d * lax.rsqrt(var + 1e-6) * g_ref[...] + b_ref[...]
    out_ref[0, 0:1, :] = gt_ref[...]
    out_ref[0, 1:, :] = y


TBLK = 1024


def kernel(x, in_degree, out_degree, link, length, entity_table,
           in_deg_table, out_deg_table, rel_table, ffn_W,
           ln_gamma, ln_beta, graph_token):
    idx = x.astype(jnp.int32).reshape(T * K // 128, 128)
    ent = _ent_gather(idx, entity_table)

    link_flat = link.reshape(T, K, R)
    rel = pl.pallas_call(
        _rel_body,
        grid=(T // TBLK,),
        in_specs=[
            pl.BlockSpec((R, H), lambda i: (0, 0)),
            pl.BlockSpec((H, H), lambda i: (0, 0)),
            pl.BlockSpec((TBLK, K, R), lambda i: (i, 0, 0)),
        ],
        out_specs=pl.BlockSpec((TBLK, H), lambda i: (i, 0)),
        out_shape=jax.ShapeDtypeStruct((T, H), jnp.bfloat16),
        scratch_shapes=[pltpu.VMEM((R, H), jnp.float32)],
    )(rel_table, ffn_W, link_flat)

    g2 = ln_gamma.reshape(1, H)
    b2 = ln_beta.reshape(1, H)
    out = pl.pallas_call(
        _fin_body,
        grid=(B,),
        in_specs=[
            pl.BlockSpec((1, N, H), lambda i: (i, 0, 0)),
            pl.BlockSpec((1, N, H), lambda i: (i, 0, 0)),
            pl.BlockSpec((1, H), lambda i: (0, 0)),
            pl.BlockSpec((1, H), lambda i: (0, 0)),
            pl.BlockSpec((1, H), lambda i: (0, 0)),
        ],
        out_specs=pl.BlockSpec((1, N + 1, H), lambda i: (i, 0, 0)),
        out_shape=jax.ShapeDtypeStruct((B, N + 1, H), jnp.float32),
    )(rel.reshape(B, N, H), ent.reshape(B, N, H), graph_token, g2, b2)
    return out
